# baseline (speedup 1.0000x reference)
import functools

import jax
import jax.numpy as jnp
from jax import lax
from jax.experimental import pallas as pl
from jax.experimental.pallas import tpu as pltpu
from jax.experimental.pallas import tpu_sc as plsc

NUM_BINS = 64
F = 32
N = 1048576
NC, NS, L = 2, 16, 16
NW = NC * NS

CHUNK_V = 256
P1_V = N // NS
P1_NCHUNK = P1_V // CHUNK_V
P2_V = N // NW
P2_NCHUNK = P2_V // CHUNK_V
HIST_W = NUM_BINS * F


def _sc_hist_body(x_hbm, out_hbm, buf0, buf1, stage, allv, shared, hist_a,
                  hist_b, sem0, sem1):
    buf = [buf0, buf1]
    sem = [sem0, sem1]
    xf = x_hbm
    sid = lax.axis_index("s")
    wid = sid * NC + lax.axis_index("c")

    iota = lax.iota(jnp.int32, L)
    ones = jnp.full((L,), 1.0, jnp.float32)
    zeros = jnp.zeros((L,), jnp.float32)
    hist_g = [hist_a, hist_b]

    def zero_body(i, _):
        hist_a[pl.ds(i * L, L)] = zeros
        hist_b[pl.ds(i * L, L)] = zeros
        return 0

    lax.fori_loop(0, HIST_W // L, zero_body, 0)

    def start(row0, b):
        pltpu.async_copy(xf.at[pl.ds(row0, CHUNK_V)], buf[b], sem[b])

    def wait(b):
        pltpu.make_async_copy(xf.at[pl.ds(0, CHUNK_V)], buf[b],
                              sem[b]).wait()

    p1_base = sid * P1_V
    inf = jnp.full((L,), jnp.inf, jnp.float32)
    ninf = jnp.full((L,), -jnp.inf, jnp.float32)

    def p1_compute(b, acc):
        def row_body(r4, acc):
            accl = list(acc)
            for rr in range(4):
                r = r4 * 4 + rr
                p = rr % 2
                for g in range(2):
                    x = buf[b][r, pl.ds(g * L, L)]
                    accl[g * 4 + p] = jnp.minimum(accl[g * 4 + p], x)
                    accl[g * 4 + 2 + p] = jnp.maximum(
                        accl[g * 4 + 2 + p], x)
            return tuple(accl)

        return lax.fori_loop(0, CHUNK_V // 4, row_body, acc)

    acc = (inf, inf, ninf, ninf, inf, inf, ninf, ninf)
    start(p1_base, 0)

    def p1_outer(ch, acc):
        wait(0)

        @pl.when(ch + 1 < P1_NCHUNK)
        def _():
            start(p1_base + (ch + 1) * CHUNK_V, 1)

        acc = p1_compute(0, acc)
        wait(1)

        @pl.when(ch + 2 < P1_NCHUNK)
        def _():
            start(p1_base + (ch + 2) * CHUNK_V, 0)

        return p1_compute(1, acc)

    acc = lax.fori_loop(0, P1_NCHUNK // 2, lambda i, a: p1_outer(i * 2, a),
                        acc)

    mn_a = jnp.minimum(acc[0], acc[1])
    mx_a = jnp.maximum(acc[2], acc[3])
    mn_b = jnp.minimum(acc[4], acc[5])
    mx_b = jnp.maximum(acc[6], acc[7])

    stage[pl.ds(0, L)] = mn_a
    stage[pl.ds(L, L)] = mx_a
    stage[pl.ds(2 * L, L)] = mn_b
    stage[pl.ds(3 * L, L)] = mx_b
    pltpu.sync_copy(stage, shared.at[pl.ds(sid * 4 * L, 4 * L)])
    plsc.subcore_barrier()
    pltpu.sync_copy(shared, allv)
    for t in range(NS):
        o = t * 4 * L
        mn_a = jnp.minimum(mn_a, allv[pl.ds(o, L)])
        mx_a = jnp.maximum(mx_a, allv[pl.ds(o + L, L)])
        mn_b = jnp.minimum(mn_b, allv[pl.ds(o + 2 * L, L)])
        mx_b = jnp.maximum(mx_b, allv[pl.ds(o + 3 * L, L)])

    lo = jnp.zeros((L,), jnp.int32)
    hi = jnp.full((L,), NUM_BINS - 1, jnp.int32)
    coff = [iota, iota + L]
    params = []
    for mn, mx in ((mn_a, mx_a), (mn_b, mx_b)):
        mins = mn - 0.5
        maxs = mx + 0.5
        width = (maxs - mins) * (1.0 / NUM_BINS)
        invw = 1.0 / width
        params.append((invw, -mins * invw))

    p2_base = wid * P2_V

    def p2_compute(b):
        @plsc.parallel_loop(0, CHUNK_V // 4, unroll=2)
        def vec_body(r4):
            for rr in range(4):
                r = r4 * 4 + rr
                for g in range(2):
                    invw, bias = params[g]
                    x = buf[b][r, pl.ds(g * L, L)]
                    t = x * invw + bias
                    idx = t.astype(jnp.int32)
                    idx = jnp.minimum(jnp.maximum(idx, lo), hi)
                    addr = idx * F + coff[g]
                    plsc.addupdate_scatter(hist_g[g], [addr], ones)

    start(p2_base, 0)

    @pl.loop(0, P2_NCHUNK, step=2)
    def p2_outer(ch):
        wait(0)
        start(p2_base + (ch + 1) * CHUNK_V, 1)
        p2_compute(0)
        wait(1)

        @pl.when(ch + 2 < P2_NCHUNK)
        def _():
            start(p2_base + (ch + 2) * CHUNK_V, 0)

        p2_compute(1)

    def merge_body(i, _):
        s = pl.ds(i * L, L)
        hist_a[s] = hist_a[s] + hist_b[s]
        return 0

    lax.fori_loop(0, HIST_W // L, merge_body, 0)
    pltpu.sync_copy(hist_a, out_hbm.at[wid])


def _sc_hist(x):
    mesh = plsc.VectorSubcoreMesh(
        core_axis_name="c", subcore_axis_name="s", num_cores=NC,
        num_subcores=NS)
    run = pl.kernel(
        _sc_hist_body,
        out_type=jax.ShapeDtypeStruct((NW, HIST_W), jnp.float32),
        mesh=mesh,
        scratch_types=[
            pltpu.VMEM((CHUNK_V, F), jnp.float32),
            pltpu.VMEM((CHUNK_V, F), jnp.float32),
            pltpu.VMEM((4 * L,), jnp.float32),
            pltpu.VMEM((NS * 4 * L,), jnp.float32),
            pltpu.VMEM_SHARED((NS * 4 * L,), jnp.float32),
            pltpu.VMEM((HIST_W,), jnp.float32),
            pltpu.VMEM((HIST_W,), jnp.float32),
            pltpu.SemaphoreType.DMA,
            pltpu.SemaphoreType.DMA,
        ],
        compiler_params=pltpu.CompilerParams(needs_layout_passes=False),
    )
    return run(x)


def _reduce_body(t_ref, out_ref):
    s = jnp.sum(t_ref[...], axis=0)
    out_ref[...] = s.reshape(HIST_W // 128, 128)


def _reduce_tiles(tiles):
    return pl.pallas_call(
        _reduce_body,
        out_shape=jax.ShapeDtypeStruct((HIST_W // 128, 128), jnp.float32),
    )(tiles)


@jax.jit
def kernel(inputs):
    tiles = _sc_hist(inputs)
    counts = _reduce_tiles(tiles)
    return counts.reshape(NUM_BINS, F)


# speedup vs baseline: 1.0012x; 1.0012x over previous
import functools

import jax
import jax.numpy as jnp
from jax import lax
from jax.experimental import pallas as pl
from jax.experimental.pallas import tpu as pltpu
from jax.experimental.pallas import tpu_sc as plsc

NUM_BINS = 64
F = 32
N = 1048576
NC, NS, L = 2, 16, 16
NW = NC * NS

CHUNK_V = 256
P1_V = N // NS
P1_NCHUNK = P1_V // CHUNK_V
P2_V = N // NW
P2_NCHUNK = P2_V // CHUNK_V
HIST_W = NUM_BINS * F


def _sc_hist_body(x_hbm, out_hbm, buf0, buf1, stage, allv, shared, hist_a,
                  hist_b, sem0, sem1):
    buf = [buf0, buf1]
    sem = [sem0, sem1]
    xf = x_hbm
    sid = lax.axis_index("s")
    wid = sid * NC + lax.axis_index("c")

    iota = lax.iota(jnp.int32, L)
    ones = jnp.full((L,), 1.0, jnp.float32)
    zeros = jnp.zeros((L,), jnp.float32)
    hist_g = [hist_a, hist_b]

    def zero_body(i, _):
        hist_a[pl.ds(i * L, L)] = zeros
        hist_b[pl.ds(i * L, L)] = zeros
        return 0

    lax.fori_loop(0, HIST_W // L, zero_body, 0)

    def start(row0, b):
        pltpu.async_copy(xf.at[pl.ds(row0, CHUNK_V)], buf[b], sem[b])

    def wait(b):
        pltpu.make_async_copy(xf.at[pl.ds(0, CHUNK_V)], buf[b],
                              sem[b]).wait()

    p1_base = sid * P1_V
    inf = jnp.full((L,), jnp.inf, jnp.float32)
    ninf = jnp.full((L,), -jnp.inf, jnp.float32)

    def p1_compute(b, acc):
        @plsc.parallel_loop(0, CHUNK_V // 4, unroll=2, carry=acc)
        def row_body(r4, acc):
            accl = list(acc)
            for rr in range(4):
                r = r4 * 4 + rr
                p = rr % 2
                for g in range(2):
                    x = buf[b][r, pl.ds(g * L, L)]
                    accl[g * 4 + p] = jnp.minimum(accl[g * 4 + p], x)
                    accl[g * 4 + 2 + p] = jnp.maximum(
                        accl[g * 4 + 2 + p], x)
            return tuple(accl)

        return row_body

    acc = (inf, inf, ninf, ninf, inf, inf, ninf, ninf)
    start(p1_base, 0)

    def p1_outer(ch, acc):
        wait(0)

        @pl.when(ch + 1 < P1_NCHUNK)
        def _():
            start(p1_base + (ch + 1) * CHUNK_V, 1)

        acc = p1_compute(0, acc)
        wait(1)

        @pl.when(ch + 2 < P1_NCHUNK)
        def _():
            start(p1_base + (ch + 2) * CHUNK_V, 0)

        return p1_compute(1, acc)

    acc = lax.fori_loop(0, P1_NCHUNK // 2, lambda i, a: p1_outer(i * 2, a),
                        acc)

    mn_a = jnp.minimum(acc[0], acc[1])
    mx_a = jnp.maximum(acc[2], acc[3])
    mn_b = jnp.minimum(acc[4], acc[5])
    mx_b = jnp.maximum(acc[6], acc[7])

    stage[pl.ds(0, L)] = mn_a
    stage[pl.ds(L, L)] = mx_a
    stage[pl.ds(2 * L, L)] = mn_b
    stage[pl.ds(3 * L, L)] = mx_b
    pltpu.sync_copy(stage, shared.at[pl.ds(sid * 4 * L, 4 * L)])
    plsc.subcore_barrier()
    pltpu.sync_copy(shared, allv)
    for t in range(NS):
        o = t * 4 * L
        mn_a = jnp.minimum(mn_a, allv[pl.ds(o, L)])
        mx_a = jnp.maximum(mx_a, allv[pl.ds(o + L, L)])
        mn_b = jnp.minimum(mn_b, allv[pl.ds(o + 2 * L, L)])
        mx_b = jnp.maximum(mx_b, allv[pl.ds(o + 3 * L, L)])

    lo = jnp.zeros((L,), jnp.int32)
    hi = jnp.full((L,), NUM_BINS - 1, jnp.int32)
    coff = [iota, iota + L]
    params = []
    for mn, mx in ((mn_a, mx_a), (mn_b, mx_b)):
        mins = mn - 0.5
        maxs = mx + 0.5
        width = (maxs - mins) * (1.0 / NUM_BINS)
        invw = 1.0 / width
        params.append((invw, -mins * invw))

    p2_base = wid * P2_V

    def p2_compute(b):
        @plsc.parallel_loop(0, CHUNK_V // 4, unroll=2)
        def vec_body(r4):
            for rr in range(4):
                r = r4 * 4 + rr
                for g in range(2):
                    invw, bias = params[g]
                    x = buf[b][r, pl.ds(g * L, L)]
                    t = x * invw + bias
                    idx = t.astype(jnp.int32)
                    idx = jnp.minimum(jnp.maximum(idx, lo), hi)
                    addr = idx * F + coff[g]
                    plsc.addupdate_scatter(hist_g[g], [addr], ones)

    start(p2_base, 0)

    @pl.loop(0, P2_NCHUNK, step=2)
    def p2_outer(ch):
        wait(0)
        start(p2_base + (ch + 1) * CHUNK_V, 1)
        p2_compute(0)
        wait(1)

        @pl.when(ch + 2 < P2_NCHUNK)
        def _():
            start(p2_base + (ch + 2) * CHUNK_V, 0)

        p2_compute(1)

    def merge_body(i, _):
        s = pl.ds(i * L, L)
        hist_a[s] = hist_a[s] + hist_b[s]
        return 0

    lax.fori_loop(0, HIST_W // L, merge_body, 0)
    pltpu.sync_copy(hist_a, out_hbm.at[wid])


def _sc_hist(x):
    mesh = plsc.VectorSubcoreMesh(
        core_axis_name="c", subcore_axis_name="s", num_cores=NC,
        num_subcores=NS)
    run = pl.kernel(
        _sc_hist_body,
        out_type=jax.ShapeDtypeStruct((NW, HIST_W), jnp.float32),
        mesh=mesh,
        scratch_types=[
            pltpu.VMEM((CHUNK_V, F), jnp.float32),
            pltpu.VMEM((CHUNK_V, F), jnp.float32),
            pltpu.VMEM((4 * L,), jnp.float32),
            pltpu.VMEM((NS * 4 * L,), jnp.float32),
            pltpu.VMEM_SHARED((NS * 4 * L,), jnp.float32),
            pltpu.VMEM((HIST_W,), jnp.float32),
            pltpu.VMEM((HIST_W,), jnp.float32),
            pltpu.SemaphoreType.DMA,
            pltpu.SemaphoreType.DMA,
        ],
        compiler_params=pltpu.CompilerParams(needs_layout_passes=False),
    )
    return run(x)


def _reduce_body(t_ref, out_ref):
    s = jnp.sum(t_ref[...], axis=0)
    out_ref[...] = s.reshape(HIST_W // 128, 128)


def _reduce_tiles(tiles):
    return pl.pallas_call(
        _reduce_body,
        out_shape=jax.ShapeDtypeStruct((HIST_W // 128, 128), jnp.float32),
    )(tiles)


@jax.jit
def kernel(inputs):
    tiles = _sc_hist(inputs)
    counts = _reduce_tiles(tiles)
    return counts.reshape(NUM_BINS, F)


# speedup vs baseline: 1.4684x; 1.4666x over previous
import functools

import jax
import jax.numpy as jnp
from jax import lax
from jax.experimental import pallas as pl
from jax.experimental.pallas import tpu as pltpu
from jax.experimental.pallas import tpu_sc as plsc

NUM_BINS = 64
F = 32
N = 1048576
NC, NS, L = 2, 16, 16
NW = NC * NS
NROW128 = N // 4
ROWS_W = NROW128 // NW
CHUNK_ROWS = 256
CHUNK_W = CHUNK_ROWS * 128
NCHUNK = ROWS_W // CHUNK_ROWS
UNROLL = 8
HIST_W = NUM_BINS * F


def _minmax_body(x_ref, min_ref, max_ref):
    i = pl.program_id(0)
    bmin = jnp.min(x_ref[...], axis=0, keepdims=True)
    bmax = jnp.max(x_ref[...], axis=0, keepdims=True)

    @pl.when(i == 0)
    def _():
        min_ref[...] = bmin
        max_ref[...] = bmax

    @pl.when(i > 0)
    def _():
        min_ref[...] = jnp.minimum(min_ref[...], bmin)
        max_ref[...] = jnp.maximum(max_ref[...], bmax)


def _colwise_minmax(x128):
    rows = x128.shape[0]
    block_rows = 2048
    grid = rows // block_rows
    mn, mx = pl.pallas_call(
        _minmax_body,
        grid=(grid,),
        in_specs=[pl.BlockSpec((block_rows, 128), lambda i: (i, 0))],
        out_specs=[
            pl.BlockSpec((1, 128), lambda i: (0, 0)),
            pl.BlockSpec((1, 128), lambda i: (0, 0)),
        ],
        out_shape=[
            jax.ShapeDtypeStruct((1, 128), jnp.float32),
            jax.ShapeDtypeStruct((1, 128), jnp.float32),
        ],
    )(x128)
    return mn, mx


def _sc_hist_body(x_hbm, invw_hbm, bias2_hbm, out_hbm, buf0, buf1, cvec,
                  hist_a, hist_b, hist_c, hist_d, sem0, sem1):
    buf = [buf0, buf1]
    sem = [sem0, sem1]
    wid = lax.axis_index("s") * NC + lax.axis_index("c")
    base = wid * ROWS_W

    pltpu.sync_copy(invw_hbm, cvec.at[pl.ds(0, F)])
    pltpu.sync_copy(bias2_hbm, cvec.at[pl.ds(F, F)])
    invw_g = [cvec[pl.ds(0, L)], cvec[pl.ds(L, L)]]
    bias2_g = [cvec[pl.ds(F, L)], cvec[pl.ds(F + L, L)]]
    iota = lax.iota(jnp.int32, L)
    lo_g = [iota * NUM_BINS, (iota + L) * NUM_BINS]
    hi_g = [lo_g[0] + (NUM_BINS - 1), lo_g[1] + (NUM_BINS - 1)]
    ones = jnp.full((L,), 1.0, jnp.float32)
    zeros = jnp.zeros((L,), jnp.float32)
    hist_g = [hist_a, hist_b, hist_c, hist_d]

    def zero_body(i, _):
        for h in hist_g:
            h[pl.ds(i * L, L)] = zeros
        return 0

    lax.fori_loop(0, HIST_W // L, zero_body, 0)

    def compute(b):
        @plsc.parallel_loop(0, CHUNK_ROWS, unroll=2)
        def vec_body(r):
            for k in range(8):
                g = k % 2
                x = buf[b][r, pl.ds(k * L, L)]
                t = x * invw_g[g] + bias2_g[g]
                idx = t.astype(jnp.int32)
                idx = jnp.minimum(jnp.maximum(idx, lo_g[g]), hi_g[g])
                plsc.addupdate_scatter(hist_g[k % 4], [idx], ones)

    def start(ch, b):
        pltpu.async_copy(
            x_hbm.at[pl.ds(base + ch * CHUNK_ROWS, CHUNK_ROWS)], buf[b],
            sem[b])

    def wait(b):
        pltpu.make_async_copy(x_hbm.at[pl.ds(0, CHUNK_ROWS)], buf[b],
                              sem[b]).wait()

    start(0, 0)

    @pl.loop(0, NCHUNK, step=2)
    def chunk_body(ch):
        wait(0)
        start(ch + 1, 1)
        compute(0)
        wait(1)

        @pl.when(ch + 2 < NCHUNK)
        def _():
            start(ch + 2, 0)

        compute(1)

    def merge_body(i, _):
        s = pl.ds(i * L, L)
        hist_a[s] = (hist_a[s] + hist_b[s]) + (hist_c[s] + hist_d[s])
        return 0

    lax.fori_loop(0, HIST_W // L, merge_body, 0)
    pltpu.sync_copy(hist_a, out_hbm.at[wid])


def _sc_hist(x_flat, invw, bias2):
    mesh = plsc.VectorSubcoreMesh(
        core_axis_name="c", subcore_axis_name="s", num_cores=NC,
        num_subcores=NS)
    run = pl.kernel(
        _sc_hist_body,
        out_type=jax.ShapeDtypeStruct((NW, HIST_W), jnp.float32),
        mesh=mesh,
        scratch_types=[
            pltpu.VMEM((CHUNK_ROWS, 128), jnp.float32),
            pltpu.VMEM((CHUNK_ROWS, 128), jnp.float32),
            pltpu.VMEM((2 * F,), jnp.float32),
            pltpu.VMEM((HIST_W,), jnp.float32),
            pltpu.VMEM((HIST_W,), jnp.float32),
            pltpu.VMEM((HIST_W,), jnp.float32),
            pltpu.VMEM((HIST_W,), jnp.float32),
            pltpu.SemaphoreType.DMA,
            pltpu.SemaphoreType.DMA,
        ],
        compiler_params=pltpu.CompilerParams(
            needs_layout_passes=False, use_tc_tiling_on_sc=True),
    )
    return run(x_flat, invw, bias2)


@jax.jit
def kernel(inputs):
    x128 = inputs.reshape(N // 4, 128)
    mn, mx = _colwise_minmax(x128)
    col_min = jnp.min(mn.reshape(4, F), axis=0)
    col_max = jnp.max(mx.reshape(4, F), axis=0)
    mins = col_min - 0.5
    maxs = col_max + 0.5
    width = (maxs - mins) / NUM_BINS
    invw = 1.0 / width
    col = jnp.arange(F, dtype=jnp.float32)
    bias2 = -mins * invw + col * NUM_BINS

    tiles = _sc_hist(x128, invw, bias2)
    counts = tiles.sum(axis=0).reshape(F, NUM_BINS)
    return counts.T
